# baseline (speedup 1.0000x reference)
import functools

import jax
import jax.numpy as jnp
from jax.experimental import pallas as pl
from jax.experimental.pallas import tpu as pltpu

K = 128
BLK = 128
BIG = 1e30
NEG = -3e38
F32 = jnp.float32


def _elu(v):
    return jnp.where(v > 0.0, v, jnp.exp(jnp.minimum(v, 0.0)) - 1.0)


def _dot(a, b, prec=jax.lax.Precision.HIGHEST):
    return jax.lax.dot_general(a, b, (((1,), (0,)), ((), ())),
                               precision=prec,
                               preferred_element_type=F32)


def _split(a):
    hi = a.astype(jnp.bfloat16)
    lo = (a - hi.astype(F32)).astype(jnp.bfloat16)
    return hi, lo


def _dot3(ahi, alo, b):
    bhi, blo = _split(b)
    d = jax.lax.Precision.DEFAULT
    return _dot(ahi, bhi, d) + (_dot(ahi, blo, d) + _dot(alo, bhi, d))


def _prep1_body(x_ref, w0_ref, b0_ref, w1_ref, b1_ref, w2_ref, b2_ref,
                ws_ref, bs_ref, wh_ref, bh_ref, xe_ref, s_ref, h_ref):
    x = x_ref[...]
    h0 = _elu(_dot(x, w0_ref[...]) + b0_ref[0:1, :])
    h1 = _elu(_dot(h0, w1_ref[...]) + b1_ref[0:1, :])
    xe = _dot(h1, w2_ref[...]) + b2_ref[0:1, :]
    xe_ref[...] = xe
    s_ref[...] = _dot(xe, ws_ref[...]) + bs_ref[0:1, :]
    h_ref[...] = _dot(xe, wh_ref[...]) + bh_ref[0:1, :]


def _proj2_body(xa_ref, xb_ref, wsa_ref, wsb_ref, bs_ref,
                wha_ref, whb_ref, bh_ref, s_ref, h_ref):
    xa = xa_ref[...]
    xb = xb_ref[...]
    s_ref[...] = _dot(xa, wsa_ref[...]) + _dot(xb, wsb_ref[...]) \
        + bs_ref[0:1, :]
    h_ref[...] = _dot(xa, wha_ref[...]) + _dot(xb, whb_ref[...]) \
        + bh_ref[0:1, :]


def _conv_body(nx, *refs):
    (info_ref, qs_ref, sT_ref, h_ref, hT_ref, brow_ref, bcol_ref) = refs[:7]
    x_refs = refs[7:7 + nx]
    wo1_refs = refs[7 + nx:7 + 2 * nx]
    wo2_ref, bo2_ref, out_ref, d2_ref = refs[7 + 2 * nx:]
    i = pl.program_id(0)
    col_lo = info_ref[0, i]
    n_t = info_ref[1, i]
    Q = qs_ref[...]
    qn = jnp.sum(Q * Q, axis=1, keepdims=True)
    qb = bcol_ref[...]
    Qhi, Qlo = _split(Q)

    def dist_tile(t, lane_min):
        c0 = pl.multiple_of(col_lo + t * BLK, BLK)
        St = sT_ref[:, pl.ds(c0, BLK)]
        sn = jnp.sum(St * St, axis=0, keepdims=True)
        cb = brow_ref[0:1, pl.ds(c0, BLK)]
        d = qn + sn - 2.0 * _dot3(Qhi, Qlo, St)
        d = jnp.maximum(d, 0.0)
        d = jnp.where(qb == cb, d, BIG)
        d2_ref[:, pl.ds(t * BLK, BLK)] = d
        return jnp.minimum(lane_min, d)

    lane_min = jax.lax.fori_loop(0, n_t, dist_tile,
                                 jnp.full((BLK, BLK), BIG, F32))

    def count(pred_thr, strict):
        def cbody(tt, acc):
            dd = d2_ref[:, pl.ds(tt * BLK, BLK)]
            m = (dd < pred_thr) if strict else (dd <= pred_thr)
            return acc + jnp.where(m, 1.0, 0.0)
        acc = jax.lax.fori_loop(0, n_t, cbody, jnp.zeros((BLK, BLK), F32))
        return jnp.sum(acc, axis=1, keepdims=True)

    def bs_body(_, carry):
        lo, hi = carry
        mid = lo + (hi - lo) // 2
        thr = jax.lax.bitcast_convert_type(jnp.maximum(mid, 0), F32)
        cnt = count(thr, strict=False)
        sel = (cnt >= K) & (mid > lo)
        hi = jnp.where(sel, mid, hi)
        lo = jnp.where(sel, lo, mid)
        return lo, hi

    hi0 = jax.lax.bitcast_convert_type(
        jnp.max(lane_min, axis=1, keepdims=True), jnp.int32)
    r0 = (K + n_t - 1) // n_t

    def mini_body(_, carry):
        lo, hi = carry
        mid = lo + (hi - lo) // 2
        thr = jax.lax.bitcast_convert_type(jnp.maximum(mid, 0), F32)
        cnt = jnp.sum(jnp.where(lane_min <= thr, 1.0, 0.0),
                      axis=1, keepdims=True)
        sel = (cnt >= r0.astype(F32)) & (mid > lo)
        return jnp.where(sel, lo, mid), jnp.where(sel, mid, hi)

    lo0 = jnp.full((BLK, 1), -1, jnp.int32)
    lo0, _ = jax.lax.fori_loop(0, 16, mini_body, (lo0, hi0))

    def bs_cond(carry):
        lo, hi = carry
        return jnp.max(hi - lo) > 1

    lo_f, hi_f = jax.lax.while_loop(bs_cond, lambda c: bs_body(0, c),
                                    (lo0, hi0))
    tstar = jax.lax.bitcast_convert_type(hi_f, F32)

    def cbody2(tt, carry):
        a_le, a_lt = carry
        dd = d2_ref[:, pl.ds(tt * BLK, BLK)]
        return (a_le + jnp.where(dd <= tstar, 1.0, 0.0),
                a_lt + jnp.where(dd < tstar, 1.0, 0.0))
    z2 = jnp.zeros((BLK, BLK), F32)
    a_le, a_lt = jax.lax.fori_loop(0, n_t, cbody2, (z2, z2))
    c_le = jnp.sum(a_le, axis=1, keepdims=True)
    budget = K - jnp.sum(a_lt, axis=1, keepdims=True)

    def agg_mean_max(sel_fn):
        def agg_body(t, carry):
            acc_mean, maxs, tie_seen = carry
            c0 = pl.multiple_of(col_lo + t * BLK, BLK)
            dd = d2_ref[:, pl.ds(t * BLK, BLK)]
            sel, tie_seen = sel_fn(dd, tie_seen)
            w = jnp.where(sel, jnp.exp(-10.0 * dd), 0.0)
            Ht = h_ref[pl.ds(c0, BLK), :]
            whi, wlo = _split(w)
            acc_mean = acc_mean + _dot3(whi, wlo, Ht)
            new_maxs = []
            for c in range(8):
                hc = hT_ref[c:c + 1, pl.ds(c0, BLK)]
                cand = jnp.where(sel, w * hc, NEG)
                new_maxs.append(jnp.maximum(
                    maxs[c], jnp.max(cand, axis=1, keepdims=True)))
            return acc_mean, tuple(new_maxs), tie_seen

        acc0 = jnp.zeros((BLK, 8), F32)
        maxs0 = tuple(jnp.full((BLK, 1), NEG, F32) for _ in range(8))
        ts0 = jnp.zeros((BLK, 1), F32)
        return jax.lax.fori_loop(0, n_t, agg_body, (acc0, maxs0, ts0))

    def sel_fast(dd, tie_seen):
        return dd <= tstar, tie_seen

    row_i = jax.lax.broadcasted_iota(jnp.int32, (BLK, BLK), 0)
    col_i = jax.lax.broadcasted_iota(jnp.int32, (BLK, BLK), 1)
    tri = (row_i <= col_i).astype(F32)

    def sel_tie(dd, tie_seen):
        lt = dd < tstar
        eq = dd == tstar
        eqf = eq.astype(F32)
        pre = _dot(eqf, tri, jax.lax.Precision.DEFAULT)
        sel = lt | (eq & ((tie_seen + pre - 1.0) < budget))
        return sel, tie_seen + jnp.sum(eqf, axis=1, keepdims=True)

    acc_mean, maxs, _ = jax.lax.cond(
        jnp.all(c_le <= float(K)),
        lambda: agg_mean_max(sel_fast),
        lambda: agg_mean_max(sel_tie))

    agg = jnp.concatenate([acc_mean * (1.0 / K)] + list(maxs), axis=1)
    y = _dot(agg, wo2_ref[...]) + bo2_ref[0:1, :]
    for xr, wr in zip(x_refs, wo1_refs):
        y = y + _dot(xr[...], wr[...])
    mu = jnp.mean(y, axis=1, keepdims=True)
    var = jnp.mean((y - mu) ** 2, axis=1, keepdims=True)
    out_ref[...] = (y - mu) / jnp.sqrt(var + 1e-5)


def _head_body(f1_ref, f2_ref, x_ref, fw0a_ref, fw0b_ref, fb0_ref,
               fw1_ref, fb1_ref, ow0a_ref, ow0b_ref, ob0_ref,
               ow1_ref, ob1_ref, out_ref):
    f = _elu(_dot(f1_ref[...], fw0a_ref[...])
             + _dot(f2_ref[...], fw0b_ref[...]) + fb0_ref[0:1, :])
    g = _dot(f, fw1_ref[...]) + fb1_ref[0:1, :]
    o = _elu(_dot(g, ow0a_ref[...])
             + _dot(x_ref[:, 0:12], ow0b_ref[...]) + ob0_ref[0:1, :])
    out_ref[...] = _dot(o, ow1_ref[...]) + ob1_ref[0:1, :]


def _rep(b):
    return jnp.broadcast_to(b[None, :], (8, b.shape[0]))


def _full(arr):
    nd = arr.ndim
    return pl.BlockSpec(arr.shape, lambda *a: (0,) * nd)


def _blk(arr):
    shape = (BLK,) + arr.shape[1:]
    return pl.BlockSpec(shape, lambda i: (i,) + (0,) * (arr.ndim - 1))


def _gravnet_conv(xs, s, h, info, brow, bcol, wo1ts, wo2t, bo2, nb, NP):
    sT = s.T
    hT = h.T
    nx = len(xs)
    args = [info, s, sT, h, hT, brow, bcol, *xs, *wo1ts, wo2t, bo2]
    specs = ([pl.BlockSpec(memory_space=pltpu.SMEM), _blk(s), _full(sT),
              _full(h), _full(hT), _full(brow), _blk(bcol)]
             + [_blk(x) for x in xs] + [_full(w) for w in wo1ts]
             + [_full(wo2t), _full(bo2)])
    return pl.pallas_call(
        functools.partial(_conv_body, nx),
        grid=(nb,),
        in_specs=specs,
        out_specs=pl.BlockSpec((BLK, 16), lambda i: (i, 0)),
        out_shape=jax.ShapeDtypeStruct((NP, 16), F32),
        scratch_shapes=[pltpu.VMEM((BLK, NP), F32)],
    )(*args)


def kernel(x_pfc, batch_pfc, params):
    p = params
    N = x_pfc.shape[0]
    nb = (N + BLK - 1) // BLK
    NP = nb * BLK
    batch = batch_pfc.astype(jnp.int32)

    xp = jnp.pad(x_pfc, ((0, NP - N), (0, 0)))
    last_b = batch[N - 1]
    bpad = jnp.pad(batch, (0, NP - N), constant_values=0) \
        .at[N:].set(last_b) if NP > N else batch
    brow = jnp.pad(batch.astype(F32), (0, NP - N),
                   constant_values=-1.0)[None, :]
    brow = jnp.broadcast_to(brow, (8, NP))
    bcol = bpad.astype(F32)[:, None]

    idx0 = jnp.arange(nb, dtype=jnp.int32) * BLK
    firsts = bpad[idx0]
    lasts = bpad[jnp.minimum(idx0 + BLK - 1, NP - 1)]
    col_lo = jnp.searchsorted(batch, firsts, side='left').astype(jnp.int32)
    col_hi = jnp.searchsorted(batch, lasts, side='right').astype(jnp.int32)
    col_lo = (col_lo // BLK) * BLK
    n_t = jnp.maximum((col_hi - col_lo + BLK - 1) // BLK, 1)
    info = jnp.stack([col_lo, n_t]).astype(jnp.int32)

    p1 = p['conv1']
    prep_args = (xp, p['enc_W0'].T, _rep(p['enc_b0']), p['enc_W1'].T,
                 _rep(p['enc_b1']), p['enc_W2'].T, _rep(p['enc_b2']),
                 p1['Ws'].T, _rep(p1['bs']), p1['Wh'].T, _rep(p1['bh']))
    x_enc, s1, h1 = pl.pallas_call(
        _prep1_body,
        in_specs=[_full(a) for a in prep_args],
        out_specs=(pl.BlockSpec((NP, 16), lambda: (0, 0)),
                   pl.BlockSpec((NP, 8), lambda: (0, 0)),
                   pl.BlockSpec((NP, 8), lambda: (0, 0))),
        out_shape=(jax.ShapeDtypeStruct((NP, 16), F32),
                   jax.ShapeDtypeStruct((NP, 8), F32),
                   jax.ShapeDtypeStruct((NP, 8), F32)),
    )(*prep_args)

    feats1 = _gravnet_conv([x_enc], s1, h1, info, brow, bcol,
                           [p1['Wo1'].T], p1['Wo2'].T, _rep(p1['bo2']),
                           nb, NP)

    p2 = p['conv2']
    PFC = x_pfc.shape[1]
    proj_args = (xp, feats1, p2['Ws'][:, :PFC].T, p2['Ws'][:, PFC:].T,
                 _rep(p2['bs']), p2['Wh'][:, :PFC].T, p2['Wh'][:, PFC:].T,
                 _rep(p2['bh']))
    s2, h2 = pl.pallas_call(
        _proj2_body,
        in_specs=[_full(a) for a in proj_args],
        out_specs=(pl.BlockSpec((NP, 8), lambda: (0, 0)),) * 2,
        out_shape=(jax.ShapeDtypeStruct((NP, 8), F32),) * 2,
    )(*proj_args)

    feats2 = _gravnet_conv([xp, feats1], s2, h2, info, brow, bcol,
                           [p2['Wo1'][:, :PFC].T, p2['Wo1'][:, PFC:].T],
                           p2['Wo2'].T, _rep(p2['bo2']), nb, NP)

    args = (feats1, feats2, xp, p['ffn_W0'][:, :16].T,
            p['ffn_W0'][:, 16:].T, _rep(p['ffn_b0']),
            p['ffn_W1'].T, _rep(p['ffn_b1']), p['out_W0'][:, :4].T,
            p['out_W0'][:, 4:].T, _rep(p['out_b0']), p['out_W1'].T,
            _rep(p['out_b1']))
    out = pl.pallas_call(
        _head_body,
        in_specs=[_full(a) for a in args],
        out_specs=pl.BlockSpec((NP, 1), lambda: (0, 0)),
        out_shape=jax.ShapeDtypeStruct((NP, 1), F32),
    )(*args)

    return (out[:N], batch_pfc, x_enc[:N])


# speedup vs baseline: 1.1014x; 1.1014x over previous
import functools

import jax
import jax.numpy as jnp
from jax.experimental import pallas as pl
from jax.experimental.pallas import tpu as pltpu

K = 128
BLK = 128
BIG = 1e30
NEG = -3e38
F32 = jnp.float32


def _elu(v):
    return jnp.where(v > 0.0, v, jnp.exp(jnp.minimum(v, 0.0)) - 1.0)


def _dot(a, b, prec=jax.lax.Precision.HIGHEST):
    return jax.lax.dot_general(a, b, (((1,), (0,)), ((), ())),
                               precision=prec,
                               preferred_element_type=F32)


def _split(a):
    hi = a.astype(jnp.bfloat16)
    lo = (a - hi.astype(F32)).astype(jnp.bfloat16)
    return hi, lo


def _dot3(ahi, alo, b):
    bhi, blo = _split(b)
    d = jax.lax.Precision.DEFAULT
    return _dot(ahi, bhi, d) + (_dot(ahi, blo, d) + _dot(alo, bhi, d))


def _prep1_body(x_ref, w0_ref, b0_ref, w1_ref, b1_ref, w2_ref, b2_ref,
                ws_ref, bs_ref, wh_ref, bh_ref, xe_ref, s_ref, h_ref):
    x = x_ref[...]
    h0 = _elu(_dot(x, w0_ref[...]) + b0_ref[0:1, :])
    h1 = _elu(_dot(h0, w1_ref[...]) + b1_ref[0:1, :])
    xe = _dot(h1, w2_ref[...]) + b2_ref[0:1, :]
    xe_ref[...] = xe
    s_ref[...] = _dot(xe, ws_ref[...]) + bs_ref[0:1, :]
    h_ref[...] = _dot(xe, wh_ref[...]) + bh_ref[0:1, :]


def _proj2_body(xa_ref, xb_ref, wsa_ref, wsb_ref, bs_ref,
                wha_ref, whb_ref, bh_ref, s_ref, h_ref):
    xa = xa_ref[...]
    xb = xb_ref[...]
    s_ref[...] = _dot(xa, wsa_ref[...]) + _dot(xb, wsb_ref[...]) \
        + bs_ref[0:1, :]
    h_ref[...] = _dot(xa, wha_ref[...]) + _dot(xb, whb_ref[...]) \
        + bh_ref[0:1, :]


def _conv_body(nx, *refs):
    (info_ref, qs_ref, sT_ref, h_ref, hT_ref, brow_ref, bcol_ref) = refs[:7]
    x_refs = refs[7:7 + nx]
    wo1_refs = refs[7 + nx:7 + 2 * nx]
    wo2_ref, bo2_ref, out_ref, d2_ref = refs[7 + 2 * nx:]
    i = pl.program_id(0)
    col_lo = info_ref[0, i]
    n_t = info_ref[1, i]
    Q = qs_ref[...]
    qn = jnp.sum(Q * Q, axis=1, keepdims=True)
    qb = bcol_ref[...]
    Qhi, Qlo = _split(Q)

    def dist_tile(t, _):
        c0 = pl.multiple_of(col_lo + t * BLK, BLK)
        St = sT_ref[:, pl.ds(c0, BLK)]
        sn = jnp.sum(St * St, axis=0, keepdims=True)
        cb = brow_ref[0:1, pl.ds(c0, BLK)]
        d = qn + sn - 2.0 * _dot3(Qhi, Qlo, St)
        d = jnp.maximum(d, 0.0)
        d = jnp.where(qb == cb, d, BIG)
        d2_ref[:, pl.ds(t * BLK, BLK)] = d
        return 0

    jax.lax.fori_loop(0, n_t, dist_tile, 0)

    def count(pred_thr, strict):
        def cbody(tt, acc):
            dd = d2_ref[:, pl.ds(tt * BLK, BLK)]
            m = (dd < pred_thr) if strict else (dd <= pred_thr)
            return acc + jnp.where(m, 1.0, 0.0)
        acc = jax.lax.fori_loop(0, n_t, cbody, jnp.zeros((BLK, BLK), F32))
        return jnp.sum(acc, axis=1, keepdims=True)

    def bs_body(_, carry):
        lo, hi = carry
        mid = lo + (hi - lo) // 2
        thr = jax.lax.bitcast_convert_type(jnp.maximum(mid, 0), F32)
        cnt = count(thr, strict=False)
        sel = (cnt >= K) & (mid > lo)
        hi = jnp.where(sel, mid, hi)
        lo = jnp.where(sel, lo, mid)
        return lo, hi

    lo0 = jnp.full((BLK, 1), -1, jnp.int32)
    hi0 = jnp.full((BLK, 1), 0x7F800000, jnp.int32)
    _, hi_f = jax.lax.fori_loop(0, 32, bs_body, (lo0, hi0))
    tstar = jax.lax.bitcast_convert_type(hi_f, F32)

    def cbody2(tt, carry):
        a_le, a_lt = carry
        dd = d2_ref[:, pl.ds(tt * BLK, BLK)]
        return (a_le + jnp.where(dd <= tstar, 1.0, 0.0),
                a_lt + jnp.where(dd < tstar, 1.0, 0.0))
    z2 = jnp.zeros((BLK, BLK), F32)
    a_le, a_lt = jax.lax.fori_loop(0, n_t, cbody2, (z2, z2))
    c_le = jnp.sum(a_le, axis=1, keepdims=True)
    budget = K - jnp.sum(a_lt, axis=1, keepdims=True)

    def agg_mean_max(sel_fn):
        def agg_body(t, carry):
            acc_mean, maxs, tie_seen = carry
            c0 = pl.multiple_of(col_lo + t * BLK, BLK)
            dd = d2_ref[:, pl.ds(t * BLK, BLK)]
            sel, tie_seen = sel_fn(dd, tie_seen)
            w = jnp.where(sel, jnp.exp(-10.0 * dd), 0.0)
            Ht = h_ref[pl.ds(c0, BLK), :]
            whi, wlo = _split(w)
            acc_mean = acc_mean + _dot3(whi, wlo, Ht)
            new_maxs = []
            for c in range(8):
                hc = hT_ref[c:c + 1, pl.ds(c0, BLK)]
                cand = jnp.where(sel, w * hc, NEG)
                new_maxs.append(jnp.maximum(
                    maxs[c], jnp.max(cand, axis=1, keepdims=True)))
            return acc_mean, tuple(new_maxs), tie_seen

        acc0 = jnp.zeros((BLK, 8), F32)
        maxs0 = tuple(jnp.full((BLK, 1), NEG, F32) for _ in range(8))
        ts0 = jnp.zeros((BLK, 1), F32)
        return jax.lax.fori_loop(0, n_t, agg_body, (acc0, maxs0, ts0))

    def sel_fast(dd, tie_seen):
        return dd <= tstar, tie_seen

    row_i = jax.lax.broadcasted_iota(jnp.int32, (BLK, BLK), 0)
    col_i = jax.lax.broadcasted_iota(jnp.int32, (BLK, BLK), 1)
    tri = (row_i <= col_i).astype(F32)

    def sel_tie(dd, tie_seen):
        lt = dd < tstar
        eq = dd == tstar
        eqf = eq.astype(F32)
        pre = _dot(eqf, tri, jax.lax.Precision.DEFAULT)
        sel = lt | (eq & ((tie_seen + pre - 1.0) < budget))
        return sel, tie_seen + jnp.sum(eqf, axis=1, keepdims=True)

    acc_mean, maxs, _ = jax.lax.cond(
        jnp.all(c_le <= float(K)),
        lambda: agg_mean_max(sel_fast),
        lambda: agg_mean_max(sel_tie))

    agg = jnp.concatenate([acc_mean * (1.0 / K)] + list(maxs), axis=1)
    y = _dot(agg, wo2_ref[...]) + bo2_ref[0:1, :]
    for xr, wr in zip(x_refs, wo1_refs):
        y = y + _dot(xr[...], wr[...])
    mu = jnp.mean(y, axis=1, keepdims=True)
    var = jnp.mean((y - mu) ** 2, axis=1, keepdims=True)
    out_ref[...] = (y - mu) / jnp.sqrt(var + 1e-5)


def _head_body(f1_ref, f2_ref, x_ref, fw0a_ref, fw0b_ref, fb0_ref,
               fw1_ref, fb1_ref, ow0a_ref, ow0b_ref, ob0_ref,
               ow1_ref, ob1_ref, out_ref):
    f = _elu(_dot(f1_ref[...], fw0a_ref[...])
             + _dot(f2_ref[...], fw0b_ref[...]) + fb0_ref[0:1, :])
    g = _dot(f, fw1_ref[...]) + fb1_ref[0:1, :]
    o = _elu(_dot(g, ow0a_ref[...])
             + _dot(x_ref[:, 0:12], ow0b_ref[...]) + ob0_ref[0:1, :])
    out_ref[...] = _dot(o, ow1_ref[...]) + ob1_ref[0:1, :]


def _rep(b):
    return jnp.broadcast_to(b[None, :], (8, b.shape[0]))


def _full(arr):
    nd = arr.ndim
    return pl.BlockSpec(arr.shape, lambda *a: (0,) * nd)


def _blk(arr):
    shape = (BLK,) + arr.shape[1:]
    return pl.BlockSpec(shape, lambda i: (i,) + (0,) * (arr.ndim - 1))


def _gravnet_conv(xs, s, h, info, brow, bcol, wo1ts, wo2t, bo2, nb, NP):
    sT = s.T
    hT = h.T
    nx = len(xs)
    args = [info, s, sT, h, hT, brow, bcol, *xs, *wo1ts, wo2t, bo2]
    specs = ([pl.BlockSpec(memory_space=pltpu.SMEM), _blk(s), _full(sT),
              _full(h), _full(hT), _full(brow), _blk(bcol)]
             + [_blk(x) for x in xs] + [_full(w) for w in wo1ts]
             + [_full(wo2t), _full(bo2)])
    return pl.pallas_call(
        functools.partial(_conv_body, nx),
        grid=(nb,),
        in_specs=specs,
        out_specs=pl.BlockSpec((BLK, 16), lambda i: (i, 0)),
        out_shape=jax.ShapeDtypeStruct((NP, 16), F32),
        scratch_shapes=[pltpu.VMEM((BLK, NP), F32)],
    )(*args)


def kernel(x_pfc, batch_pfc, params):
    p = params
    N = x_pfc.shape[0]
    nb = (N + BLK - 1) // BLK
    NP = nb * BLK
    batch = batch_pfc.astype(jnp.int32)

    xp = jnp.pad(x_pfc, ((0, NP - N), (0, 0)))
    last_b = batch[N - 1]
    bpad = jnp.pad(batch, (0, NP - N), constant_values=0) \
        .at[N:].set(last_b) if NP > N else batch
    brow = jnp.pad(batch.astype(F32), (0, NP - N),
                   constant_values=-1.0)[None, :]
    brow = jnp.broadcast_to(brow, (8, NP))
    bcol = bpad.astype(F32)[:, None]

    idx0 = jnp.arange(nb, dtype=jnp.int32) * BLK
    firsts = bpad[idx0]
    lasts = bpad[jnp.minimum(idx0 + BLK - 1, NP - 1)]
    col_lo = jnp.searchsorted(batch, firsts, side='left').astype(jnp.int32)
    col_hi = jnp.searchsorted(batch, lasts, side='right').astype(jnp.int32)
    col_lo = (col_lo // BLK) * BLK
    n_t = jnp.maximum((col_hi - col_lo + BLK - 1) // BLK, 1)
    info = jnp.stack([col_lo, n_t]).astype(jnp.int32)

    p1 = p['conv1']
    prep_args = (xp, p['enc_W0'].T, _rep(p['enc_b0']), p['enc_W1'].T,
                 _rep(p['enc_b1']), p['enc_W2'].T, _rep(p['enc_b2']),
                 p1['Ws'].T, _rep(p1['bs']), p1['Wh'].T, _rep(p1['bh']))
    x_enc, s1, h1 = pl.pallas_call(
        _prep1_body,
        in_specs=[_full(a) for a in prep_args],
        out_specs=(pl.BlockSpec((NP, 16), lambda: (0, 0)),
                   pl.BlockSpec((NP, 8), lambda: (0, 0)),
                   pl.BlockSpec((NP, 8), lambda: (0, 0))),
        out_shape=(jax.ShapeDtypeStruct((NP, 16), F32),
                   jax.ShapeDtypeStruct((NP, 8), F32),
                   jax.ShapeDtypeStruct((NP, 8), F32)),
    )(*prep_args)

    feats1 = _gravnet_conv([x_enc], s1, h1, info, brow, bcol,
                           [p1['Wo1'].T], p1['Wo2'].T, _rep(p1['bo2']),
                           nb, NP)

    p2 = p['conv2']
    PFC = x_pfc.shape[1]
    proj_args = (xp, feats1, p2['Ws'][:, :PFC].T, p2['Ws'][:, PFC:].T,
                 _rep(p2['bs']), p2['Wh'][:, :PFC].T, p2['Wh'][:, PFC:].T,
                 _rep(p2['bh']))
    s2, h2 = pl.pallas_call(
        _proj2_body,
        in_specs=[_full(a) for a in proj_args],
        out_specs=(pl.BlockSpec((NP, 8), lambda: (0, 0)),) * 2,
        out_shape=(jax.ShapeDtypeStruct((NP, 8), F32),) * 2,
    )(*proj_args)

    feats2 = _gravnet_conv([xp, feats1], s2, h2, info, brow, bcol,
                           [p2['Wo1'][:, :PFC].T, p2['Wo1'][:, PFC:].T],
                           p2['Wo2'].T, _rep(p2['bo2']), nb, NP)

    args = (feats1, feats2, xp, p['ffn_W0'][:, :16].T,
            p['ffn_W0'][:, 16:].T, _rep(p['ffn_b0']),
            p['ffn_W1'].T, _rep(p['ffn_b1']), p['out_W0'][:, :4].T,
            p['out_W0'][:, 4:].T, _rep(p['out_b0']), p['out_W1'].T,
            _rep(p['out_b1']))
    out = pl.pallas_call(
        _head_body,
        in_specs=[_full(a) for a in args],
        out_specs=pl.BlockSpec((NP, 1), lambda: (0, 0)),
        out_shape=jax.ShapeDtypeStruct((NP, 1), F32),
    )(*args)

    return (out[:N], batch_pfc, x_enc[:N])


# speedup vs baseline: 1.1198x; 1.0167x over previous
import functools

import jax
import jax.numpy as jnp
from jax.experimental import pallas as pl
from jax.experimental.pallas import tpu as pltpu

K = 128
BLK = 128
BIG = 1e30
NEG = -3e38
F32 = jnp.float32


def _elu(v):
    return jnp.where(v > 0.0, v, jnp.exp(jnp.minimum(v, 0.0)) - 1.0)


def _dot(a, b, prec=jax.lax.Precision.HIGHEST):
    return jax.lax.dot_general(a, b, (((1,), (0,)), ((), ())),
                               precision=prec,
                               preferred_element_type=F32)


def _split(a):
    hi = a.astype(jnp.bfloat16)
    lo = (a - hi.astype(F32)).astype(jnp.bfloat16)
    return hi, lo


def _dot3(ahi, alo, b):
    bhi, blo = _split(b)
    d = jax.lax.Precision.DEFAULT
    return _dot(ahi, bhi, d) + (_dot(ahi, blo, d) + _dot(alo, bhi, d))


def _prep1_body(x_ref, w0_ref, b0_ref, w1_ref, b1_ref, w2_ref, b2_ref,
                ws_ref, bs_ref, wh_ref, bh_ref, xe_ref, s_ref, h_ref):
    x = x_ref[...]
    h0 = _elu(_dot(x, w0_ref[...]) + b0_ref[0:1, :])
    h1 = _elu(_dot(h0, w1_ref[...]) + b1_ref[0:1, :])
    xe = _dot(h1, w2_ref[...]) + b2_ref[0:1, :]
    xe_ref[...] = xe
    s_ref[...] = _dot(xe, ws_ref[...]) + bs_ref[0:1, :]
    h_ref[...] = _dot(xe, wh_ref[...]) + bh_ref[0:1, :]


def _proj2_body(xa_ref, xb_ref, wsa_ref, wsb_ref, bs_ref,
                wha_ref, whb_ref, bh_ref, s_ref, h_ref):
    xa = xa_ref[...]
    xb = xb_ref[...]
    s_ref[...] = _dot(xa, wsa_ref[...]) + _dot(xb, wsb_ref[...]) \
        + bs_ref[0:1, :]
    h_ref[...] = _dot(xa, wha_ref[...]) + _dot(xb, whb_ref[...]) \
        + bh_ref[0:1, :]


def _conv_body(nx, *refs):
    (info_ref, qs_ref, sT_ref, h_ref, hT_ref, brow_ref, bcol_ref) = refs[:7]
    x_refs = refs[7:7 + nx]
    wo1_refs = refs[7 + nx:7 + 2 * nx]
    wo2_ref, bo2_ref, out_ref, d2_ref = refs[7 + 2 * nx:]
    i = pl.program_id(0)
    col_lo = info_ref[0, i]
    n_t = info_ref[1, i]
    Q = qs_ref[...]
    qn = jnp.sum(Q * Q, axis=1, keepdims=True)
    qb = bcol_ref[...]
    Qhi, Qlo = _split(Q)

    def dist_tile(t, _):
        c0 = pl.multiple_of(col_lo + t * BLK, BLK)
        St = sT_ref[:, pl.ds(c0, BLK)]
        sn = jnp.sum(St * St, axis=0, keepdims=True)
        cb = brow_ref[0:1, pl.ds(c0, BLK)]
        d = qn + sn - 2.0 * _dot3(Qhi, Qlo, St)
        d = jnp.maximum(d, 0.0)
        d = jnp.where(qb == cb, d, BIG)
        d2_ref[:, pl.ds(t * BLK, BLK)] = d
        return 0

    jax.lax.fori_loop(0, n_t, dist_tile, 0)

    def count(pred_thr, strict):
        def cbody(tt, acc):
            dd = d2_ref[:, pl.ds(tt * BLK, BLK)]
            m = (dd < pred_thr) if strict else (dd <= pred_thr)
            return acc + jnp.where(m, 1.0, 0.0)
        acc = jax.lax.fori_loop(0, n_t, cbody, jnp.zeros((BLK, BLK), F32))
        return jnp.sum(acc, axis=1, keepdims=True)

    def bs_body(_, carry):
        lo, hi = carry
        mid = lo + (hi - lo) // 2
        thr = jax.lax.bitcast_convert_type(jnp.maximum(mid, 0), F32)
        cnt = count(thr, strict=False)
        sel = (cnt >= K) & (mid > lo)
        hi = jnp.where(sel, mid, hi)
        lo = jnp.where(sel, lo, mid)
        return lo, hi

    lo0 = jnp.full((BLK, 1), -1, jnp.int32)
    hi0 = jnp.full((BLK, 1), 0x7F800000, jnp.int32)
    _, hi_f = jax.lax.fori_loop(0, 31, bs_body, (lo0, hi0))
    tstar = jax.lax.bitcast_convert_type(hi_f, F32)

    def cbody2(tt, carry):
        a_le, a_lt = carry
        dd = d2_ref[:, pl.ds(tt * BLK, BLK)]
        return (a_le + jnp.where(dd <= tstar, 1.0, 0.0),
                a_lt + jnp.where(dd < tstar, 1.0, 0.0))
    z2 = jnp.zeros((BLK, BLK), F32)
    a_le, a_lt = jax.lax.fori_loop(0, n_t, cbody2, (z2, z2))
    c_le = jnp.sum(a_le, axis=1, keepdims=True)
    budget = K - jnp.sum(a_lt, axis=1, keepdims=True)

    def agg_mean_max(sel_fn):
        def agg_body(t, carry):
            acc_mean, maxs, tie_seen = carry
            c0 = pl.multiple_of(col_lo + t * BLK, BLK)
            dd = d2_ref[:, pl.ds(t * BLK, BLK)]
            sel, tie_seen = sel_fn(dd, tie_seen)
            w = jnp.where(sel, jnp.exp(-10.0 * dd), 0.0)
            Ht = h_ref[pl.ds(c0, BLK), :]
            whi, wlo = _split(w)
            acc_mean = acc_mean + _dot3(whi, wlo, Ht)
            new_maxs = []
            for c in range(8):
                hc = hT_ref[c:c + 1, pl.ds(c0, BLK)]
                cand = jnp.where(sel, w * hc, NEG)
                new_maxs.append(jnp.maximum(
                    maxs[c], jnp.max(cand, axis=1, keepdims=True)))
            return acc_mean, tuple(new_maxs), tie_seen

        acc0 = jnp.zeros((BLK, 8), F32)
        maxs0 = tuple(jnp.full((BLK, 1), NEG, F32) for _ in range(8))
        ts0 = jnp.zeros((BLK, 1), F32)
        return jax.lax.fori_loop(0, n_t, agg_body, (acc0, maxs0, ts0))

    def sel_fast(dd, tie_seen):
        return dd <= tstar, tie_seen

    row_i = jax.lax.broadcasted_iota(jnp.int32, (BLK, BLK), 0)
    col_i = jax.lax.broadcasted_iota(jnp.int32, (BLK, BLK), 1)
    tri = (row_i <= col_i).astype(F32)

    def sel_tie(dd, tie_seen):
        lt = dd < tstar
        eq = dd == tstar
        eqf = eq.astype(F32)
        pre = _dot(eqf, tri, jax.lax.Precision.DEFAULT)
        sel = lt | (eq & ((tie_seen + pre - 1.0) < budget))
        return sel, tie_seen + jnp.sum(eqf, axis=1, keepdims=True)

    acc_mean, maxs, _ = jax.lax.cond(
        jnp.all(c_le <= float(K)),
        lambda: agg_mean_max(sel_fast),
        lambda: agg_mean_max(sel_tie))

    agg = jnp.concatenate([acc_mean * (1.0 / K)] + list(maxs), axis=1)
    y = _dot(agg, wo2_ref[...]) + bo2_ref[0:1, :]
    for xr, wr in zip(x_refs, wo1_refs):
        y = y + _dot(xr[...], wr[...])
    mu = jnp.mean(y, axis=1, keepdims=True)
    var = jnp.mean((y - mu) ** 2, axis=1, keepdims=True)
    out_ref[...] = (y - mu) / jnp.sqrt(var + 1e-5)


def _head_body(f1_ref, f2_ref, x_ref, fw0a_ref, fw0b_ref, fb0_ref,
               fw1_ref, fb1_ref, ow0a_ref, ow0b_ref, ob0_ref,
               ow1_ref, ob1_ref, out_ref):
    f = _elu(_dot(f1_ref[...], fw0a_ref[...])
             + _dot(f2_ref[...], fw0b_ref[...]) + fb0_ref[0:1, :])
    g = _dot(f, fw1_ref[...]) + fb1_ref[0:1, :]
    o = _elu(_dot(g, ow0a_ref[...])
             + _dot(x_ref[:, 0:12], ow0b_ref[...]) + ob0_ref[0:1, :])
    out_ref[...] = _dot(o, ow1_ref[...]) + ob1_ref[0:1, :]


def _rep(b):
    return jnp.broadcast_to(b[None, :], (8, b.shape[0]))


def _full(arr):
    nd = arr.ndim
    return pl.BlockSpec(arr.shape, lambda *a: (0,) * nd)


def _blk(arr):
    shape = (BLK,) + arr.shape[1:]
    return pl.BlockSpec(shape, lambda i: (i,) + (0,) * (arr.ndim - 1))


def _gravnet_conv(xs, s, h, info, brow, bcol, wo1ts, wo2t, bo2, nb, NP):
    sT = s.T
    hT = h.T
    nx = len(xs)
    args = [info, s, sT, h, hT, brow, bcol, *xs, *wo1ts, wo2t, bo2]
    specs = ([pl.BlockSpec(memory_space=pltpu.SMEM), _blk(s), _full(sT),
              _full(h), _full(hT), _full(brow), _blk(bcol)]
             + [_blk(x) for x in xs] + [_full(w) for w in wo1ts]
             + [_full(wo2t), _full(bo2)])
    return pl.pallas_call(
        functools.partial(_conv_body, nx),
        grid=(nb,),
        in_specs=specs,
        out_specs=pl.BlockSpec((BLK, 16), lambda i: (i, 0)),
        out_shape=jax.ShapeDtypeStruct((NP, 16), F32),
        scratch_shapes=[pltpu.VMEM((BLK, NP), F32)],
    )(*args)


def kernel(x_pfc, batch_pfc, params):
    p = params
    N = x_pfc.shape[0]
    nb = (N + BLK - 1) // BLK
    NP = nb * BLK
    batch = batch_pfc.astype(jnp.int32)

    xp = jnp.pad(x_pfc, ((0, NP - N), (0, 0)))
    last_b = batch[N - 1]
    bpad = jnp.pad(batch, (0, NP - N), constant_values=0) \
        .at[N:].set(last_b) if NP > N else batch
    brow = jnp.pad(batch.astype(F32), (0, NP - N),
                   constant_values=-1.0)[None, :]
    brow = jnp.broadcast_to(brow, (8, NP))
    bcol = bpad.astype(F32)[:, None]

    idx0 = jnp.arange(nb, dtype=jnp.int32) * BLK
    firsts = bpad[idx0]
    lasts = bpad[jnp.minimum(idx0 + BLK - 1, NP - 1)]
    col_lo = jnp.searchsorted(batch, firsts, side='left').astype(jnp.int32)
    col_hi = jnp.searchsorted(batch, lasts, side='right').astype(jnp.int32)
    col_lo = (col_lo // BLK) * BLK
    n_t = jnp.maximum((col_hi - col_lo + BLK - 1) // BLK, 1)
    info = jnp.stack([col_lo, n_t]).astype(jnp.int32)

    p1 = p['conv1']
    prep_args = (xp, p['enc_W0'].T, _rep(p['enc_b0']), p['enc_W1'].T,
                 _rep(p['enc_b1']), p['enc_W2'].T, _rep(p['enc_b2']),
                 p1['Ws'].T, _rep(p1['bs']), p1['Wh'].T, _rep(p1['bh']))
    x_enc, s1, h1 = pl.pallas_call(
        _prep1_body,
        in_specs=[_full(a) for a in prep_args],
        out_specs=(pl.BlockSpec((NP, 16), lambda: (0, 0)),
                   pl.BlockSpec((NP, 8), lambda: (0, 0)),
                   pl.BlockSpec((NP, 8), lambda: (0, 0))),
        out_shape=(jax.ShapeDtypeStruct((NP, 16), F32),
                   jax.ShapeDtypeStruct((NP, 8), F32),
                   jax.ShapeDtypeStruct((NP, 8), F32)),
    )(*prep_args)

    feats1 = _gravnet_conv([x_enc], s1, h1, info, brow, bcol,
                           [p1['Wo1'].T], p1['Wo2'].T, _rep(p1['bo2']),
                           nb, NP)

    p2 = p['conv2']
    PFC = x_pfc.shape[1]
    proj_args = (xp, feats1, p2['Ws'][:, :PFC].T, p2['Ws'][:, PFC:].T,
                 _rep(p2['bs']), p2['Wh'][:, :PFC].T, p2['Wh'][:, PFC:].T,
                 _rep(p2['bh']))
    s2, h2 = pl.pallas_call(
        _proj2_body,
        in_specs=[_full(a) for a in proj_args],
        out_specs=(pl.BlockSpec((NP, 8), lambda: (0, 0)),) * 2,
        out_shape=(jax.ShapeDtypeStruct((NP, 8), F32),) * 2,
    )(*proj_args)

    feats2 = _gravnet_conv([xp, feats1], s2, h2, info, brow, bcol,
                           [p2['Wo1'][:, :PFC].T, p2['Wo1'][:, PFC:].T],
                           p2['Wo2'].T, _rep(p2['bo2']), nb, NP)

    args = (feats1, feats2, xp, p['ffn_W0'][:, :16].T,
            p['ffn_W0'][:, 16:].T, _rep(p['ffn_b0']),
            p['ffn_W1'].T, _rep(p['ffn_b1']), p['out_W0'][:, :4].T,
            p['out_W0'][:, 4:].T, _rep(p['out_b0']), p['out_W1'].T,
            _rep(p['out_b1']))
    out = pl.pallas_call(
        _head_body,
        in_specs=[_full(a) for a in args],
        out_specs=pl.BlockSpec((NP, 1), lambda: (0, 0)),
        out_shape=jax.ShapeDtypeStruct((NP, 1), F32),
    )(*args)

    return (out[:N], batch_pfc, x_enc[:N])
